# baseline (speedup 1.0000x reference)
import jax
import jax.numpy as jnp
from jax import lax
from jax.experimental import pallas as pl
from jax.experimental.pallas import tpu as pltpu
from jax.experimental.pallas import tpu_sc as plsc

NUM_CORES = 2
NUM_SUBCORES = 16
NUM_WORKERS = NUM_CORES * NUM_SUBCORES
LANES = 16
BATCH_TOTAL = 16384
ROWS_PER_W = BATCH_TOTAL // NUM_WORKERS
CHUNK = 128
NCHUNK = ROWS_PER_W // CHUNK
EMBED = 64


def _sc_body(users_hbm, items_hbm, gu_hbm, bu_hbm, gi_hbm, bi_hbm, b_hbm,
             out_hbm,
             idx_u, idx_i, gu_v, gi_v, bu_v, bi_v, b_v, out_v, sem):
    wid = lax.axis_index("s") * NUM_CORES + lax.axis_index("c")
    base = wid * ROWS_PER_W

    pltpu.sync_copy(users_hbm.at[pl.ds(wid * NCHUNK, NCHUNK)], idx_u)
    pltpu.sync_copy(items_hbm.at[pl.ds(wid * NCHUNK, NCHUNK)], idx_i)
    pltpu.sync_copy(b_hbm, b_v)

    copies = []
    for j in range(NCHUNK):
        sl = pl.ds(j * CHUNK, CHUNK)
        copies.append(pltpu.async_copy(gu_hbm.at[idx_u.at[j]], gu_v.at[sl], sem))
        copies.append(pltpu.async_copy(gi_hbm.at[idx_i.at[j]], gi_v.at[sl], sem))
        copies.append(pltpu.async_copy(bu_hbm.at[idx_u.at[j]], bu_v.at[sl], sem))
        copies.append(pltpu.async_copy(bi_hbm.at[idx_i.at[j]], bi_v.at[sl], sem))
    for c in copies:
        c.wait()

    bvec = b_v[...]
    lane = lax.iota(jnp.int32, LANES)

    def rot(x, s):
        return x.at[(lane + s) % LANES].get(mode="promise_in_bounds")

    def group(g, carry):
        sums = jnp.zeros((LANES,), jnp.float32)
        for l in range(LANES):
            r = g * LANES + l
            acc = jnp.zeros((LANES,), jnp.float32)
            for v in range(EMBED // LANES):
                sl = pl.ds(v * LANES, LANES)
                acc = acc + gu_v[r, sl] * gi_v[r, sl]
            for s in (8, 4, 2, 1):
                acc = acc + rot(acc, s)
            sums = jnp.where(lane == l, acc, sums)
        sl = pl.ds(g * LANES, LANES)
        out_v[sl] = sums + bu_v[sl] + bi_v[sl] + bvec
        return carry

    lax.fori_loop(0, ROWS_PER_W // LANES, group, 0)

    pltpu.sync_copy(out_v, out_hbm.at[pl.ds(base, ROWS_PER_W)])


def kernel(users, items, Gu, Bu, Gi, Bi, B):
    u2d = users.reshape(BATCH_TOTAL // CHUNK, CHUNK)
    i2d = items.reshape(BATCH_TOTAL // CHUNK, CHUNK)
    b16 = jnp.broadcast_to(B, (LANES,))
    mesh = plsc.VectorSubcoreMesh(core_axis_name="c", subcore_axis_name="s")
    fn = pl.kernel(
        _sc_body,
        mesh=mesh,
        compiler_params=pltpu.CompilerParams(use_tc_tiling_on_sc=False),
        out_type=jax.ShapeDtypeStruct((BATCH_TOTAL,), jnp.float32),
        scratch_types=[
            pltpu.VMEM((NCHUNK, CHUNK), jnp.int32),
            pltpu.VMEM((NCHUNK, CHUNK), jnp.int32),
            pltpu.VMEM((ROWS_PER_W, EMBED), jnp.float32),
            pltpu.VMEM((ROWS_PER_W, EMBED), jnp.float32),
            pltpu.VMEM((ROWS_PER_W,), jnp.float32),
            pltpu.VMEM((ROWS_PER_W,), jnp.float32),
            pltpu.VMEM((LANES,), jnp.float32),
            pltpu.VMEM((ROWS_PER_W,), jnp.float32),
            pltpu.SemaphoreType.DMA,
        ],
    )
    return fn(u2d, i2d, Gu, Bu, Gi, Bi, b16)
